# baseline (speedup 1.0000x reference)
import functools

import jax
import jax.numpy as jnp
from jax import lax
from jax.experimental import pallas as pl
from jax.experimental.pallas import tpu as pltpu
from jax.experimental.pallas import tpu_sc as plsc

_NC = 2
_NS = 16
_NW = _NC * _NS
_LANES = 16


def _make_gather_sum(B, L, EMB):
  assert B % _NW == 0 and EMB % _LANES == 0
  bpw = B // _NW
  chunk = L // 2
  assert chunk * 2 == L and chunk <= 128
  nvec = EMB // _LANES
  mesh = plsc.VectorSubcoreMesh(core_axis_name="c", subcore_axis_name="s")

  @functools.partial(
      pl.kernel,
      mesh=mesh,
      out_type=jax.ShapeDtypeStruct((B, EMB), jnp.float32),
      scratch_types=[
          pltpu.VMEM((bpw, 2, chunk), jnp.int32),
          pltpu.VMEM((chunk, EMB), jnp.float32),
          pltpu.VMEM((chunk, EMB), jnp.float32),
          pltpu.VMEM((bpw, EMB), jnp.float32),
          pltpu.SemaphoreType.DMA,
          pltpu.SemaphoreType.DMA,
      ],
      compiler_params=pltpu.CompilerParams(use_tc_tiling_on_sc=False),
  )
  def gather_sum(idx_hbm, table_hbm, sums_hbm, idx_v, buf0, buf1, outv,
                 sem0, sem1):
    wid = lax.axis_index("s") * _NC + lax.axis_index("c")
    base = wid * bpw
    pltpu.sync_copy(idx_hbm.at[pl.ds(base, bpw)], idx_v)

    def row(i, carry):
      cp0 = pltpu.async_copy(table_hbm.at[idx_v.at[i, 0]], buf0, sem0)
      cp1 = pltpu.async_copy(table_hbm.at[idx_v.at[i, 1]], buf1, sem1)
      cp0.wait()
      cp1.wait()

      zero = jnp.zeros((_LANES,), jnp.float32)

      def red(r, acc):
        return tuple(
            acc[k]
            + buf0[r, pl.ds(_LANES * k, _LANES)]
            + buf1[r, pl.ds(_LANES * k, _LANES)]
            for k in range(nvec))

      acc = lax.fori_loop(0, chunk, red, (zero,) * nvec)
      for k in range(nvec):
        outv[i, pl.ds(_LANES * k, _LANES)] = acc[k]
      return carry

    lax.fori_loop(0, bpw, row, 0)
    pltpu.sync_copy(outv, sums_hbm.at[pl.ds(base, bpw)])

  return gather_sum


def _make_dense(TB, B, EMB, IMG, HID, blk=512):
  assert TB % blk == 0 and B % blk == 0
  grid = (TB // blk,)
  nrep = B // blk

  def body(sums_ref, img_ref, w1e_ref, w1i_ref, b1_ref, w2_ref, b2_ref,
           out_ref):
    x = jnp.dot(sums_ref[...], w1e_ref[...],
                preferred_element_type=jnp.float32)
    x = x + jnp.dot(img_ref[...], w1i_ref[...],
                    preferred_element_type=jnp.float32)
    h = jnp.tanh(x + b1_ref[...])
    o = jnp.sum(h * w2_ref[...], axis=1) + b2_ref[0, 0]
    out_ref[...] = o.reshape(1, blk)

  return pl.pallas_call(
      body,
      grid=grid,
      in_specs=[
          pl.BlockSpec((blk, EMB), lambda g: (g % nrep, 0)),
          pl.BlockSpec((blk, IMG), lambda g: (g, 0)),
          pl.BlockSpec((EMB, HID), lambda g: (0, 0)),
          pl.BlockSpec((IMG, HID), lambda g: (0, 0)),
          pl.BlockSpec((1, HID), lambda g: (0, 0)),
          pl.BlockSpec((1, HID), lambda g: (0, 0)),
          pl.BlockSpec((1, 1), lambda g: (0, 0), memory_space=pltpu.SMEM),
      ],
      out_specs=pl.BlockSpec((1, blk), lambda g: (0, g)),
      out_shape=jax.ShapeDtypeStruct((1, TB), jnp.float32),
      compiler_params=pltpu.CompilerParams(
          dimension_semantics=("arbitrary",)),
  )


@jax.jit
def kernel(inputs, img_feat, emb_table, W1, b1, W2, b2):
  B, L = inputs.shape
  EMB = emb_table.shape[1]
  TB, IMG = img_feat.shape
  HID = W1.shape[1]

  gather_sum = _make_gather_sum(B, L, EMB)
  sums = gather_sum(inputs.reshape(B, 2, L // 2), emb_table)

  dense = _make_dense(TB, B, EMB, IMG, HID)
  out = dense(sums, img_feat, W1[:EMB], W1[EMB:], b1.reshape(1, HID),
              W2.reshape(1, HID), b2.reshape(1, 1))
  return out


# speedup vs baseline: 1.1571x; 1.1571x over previous
import functools

import jax
import jax.numpy as jnp
from jax import lax
from jax.experimental import pallas as pl
from jax.experimental.pallas import tpu as pltpu
from jax.experimental.pallas import tpu_sc as plsc

_NC = 2
_NS = 16
_NW = _NC * _NS
_LANES = 16


_NBUF = 4
_UNROLL = 8


def _make_gather_sum(B, L, EMB):
  assert B % _NW == 0 and EMB % _LANES == 0
  bpw = B // _NW
  chunk = L // 2
  assert chunk * 2 == L and chunk <= 128
  assert L % _UNROLL == 0 and bpw % _NBUF == 0
  nvec = EMB // _LANES
  mesh = plsc.VectorSubcoreMesh(core_axis_name="c", subcore_axis_name="s")

  @functools.partial(
      pl.kernel,
      mesh=mesh,
      out_type=jax.ShapeDtypeStruct((B, EMB), jnp.float32),
      scratch_types=[
          pltpu.VMEM((bpw, 2, chunk), jnp.int32),
          [pltpu.VMEM((L, EMB), jnp.float32) for _ in range(_NBUF)],
          pltpu.VMEM((bpw, EMB), jnp.float32),
          [pltpu.SemaphoreType.DMA for _ in range(_NBUF)],
      ],
      compiler_params=pltpu.CompilerParams(use_tc_tiling_on_sc=False),
  )
  def gather_sum(idx_hbm, table_hbm, sums_hbm, idx_v, bufs, outv, sems):
    wid = lax.axis_index("s") * _NC + lax.axis_index("c")
    base = wid * bpw
    pltpu.sync_copy(idx_hbm.at[pl.ds(base, bpw)], idx_v)

    def issue(j, b):
      pltpu.async_copy(table_hbm.at[idx_v.at[j, 0]],
                       bufs[b].at[pl.ds(0, chunk)], sems[b])
      pltpu.async_copy(table_hbm.at[idx_v.at[j, 1]],
                       bufs[b].at[pl.ds(chunk, chunk)], sems[b])

    def drain(j, b):
      pltpu.make_async_copy(table_hbm.at[idx_v.at[j, 0]],
                            bufs[b].at[pl.ds(0, chunk)], sems[b]).wait()
      pltpu.make_async_copy(table_hbm.at[idx_v.at[j, 1]],
                            bufs[b].at[pl.ds(chunk, chunk)], sems[b]).wait()

    for b in range(_NBUF):
      issue(b, b)

    zero = jnp.zeros((_LANES,), jnp.float32)

    def group(g, carry):
      for b in range(_NBUF):
        i = g * _NBUF + b
        drain(i, b)
        buf = bufs[b]

        def red(r, acc):
          rb = r * _UNROLL
          nxt = []
          for k in range(nvec):
            t = [buf[rb + d, pl.ds(_LANES * k, _LANES)]
                 for d in range(_UNROLL)]
            while len(t) > 1:
              t = [t[p] + t[p + 1] for p in range(0, len(t), 2)]
            nxt.append(acc[k] + t[0])
          return tuple(nxt)

        acc = lax.fori_loop(0, L // _UNROLL, red, (zero,) * nvec)
        for k in range(nvec):
          outv[i, pl.ds(_LANES * k, _LANES)] = acc[k]

        nj = i + _NBUF

        @pl.when(nj < bpw)
        def _():
          issue(nj, b)

      return carry

    lax.fori_loop(0, bpw // _NBUF, group, 0)
    pltpu.sync_copy(outv, sums_hbm.at[pl.ds(base, bpw)])

  return gather_sum


def _make_dense(TB, B, EMB, IMG, HID, blk=512):
  assert TB % blk == 0 and B % blk == 0
  grid = (TB // blk,)
  nrep = B // blk

  def body(sums_ref, img_ref, w1e_ref, w1i_ref, b1_ref, w2_ref, b2_ref,
           out_ref):
    x = jnp.dot(sums_ref[...], w1e_ref[...],
                preferred_element_type=jnp.float32)
    x = x + jnp.dot(img_ref[...], w1i_ref[...],
                    preferred_element_type=jnp.float32)
    h = jnp.tanh(x + b1_ref[...])
    o = jnp.sum(h * w2_ref[...], axis=1) + b2_ref[0, 0]
    out_ref[...] = o.reshape(1, blk)

  return pl.pallas_call(
      body,
      grid=grid,
      in_specs=[
          pl.BlockSpec((blk, EMB), lambda g: (g % nrep, 0)),
          pl.BlockSpec((blk, IMG), lambda g: (g, 0)),
          pl.BlockSpec((EMB, HID), lambda g: (0, 0)),
          pl.BlockSpec((IMG, HID), lambda g: (0, 0)),
          pl.BlockSpec((1, HID), lambda g: (0, 0)),
          pl.BlockSpec((1, HID), lambda g: (0, 0)),
          pl.BlockSpec((1, 1), lambda g: (0, 0), memory_space=pltpu.SMEM),
      ],
      out_specs=pl.BlockSpec((1, blk), lambda g: (0, g)),
      out_shape=jax.ShapeDtypeStruct((1, TB), jnp.float32),
      compiler_params=pltpu.CompilerParams(
          dimension_semantics=("arbitrary",)),
  )


@jax.jit
def kernel(inputs, img_feat, emb_table, W1, b1, W2, b2):
  B, L = inputs.shape
  EMB = emb_table.shape[1]
  TB, IMG = img_feat.shape
  HID = W1.shape[1]

  gather_sum = _make_gather_sum(B, L, EMB)
  sums = gather_sum(inputs.reshape(B, 2, L // 2), emb_table)

  dense = _make_dense(TB, B, EMB, IMG, HID)
  out = dense(sums, img_feat, W1[:EMB], W1[EMB:], b1.reshape(1, HID),
              W2.reshape(1, HID), b2.reshape(1, 1))
  return out
